# baseline (speedup 1.0000x reference)
import functools

import jax
import jax.numpy as jnp
from jax import lax
from jax.experimental import pallas as pl
from jax.experimental.pallas import tpu as pltpu
from jax.experimental.pallas import tpu_sc as plsc

_SC_CORES = 2
_SC_SUBCORES = 16
_GATHER_CHUNK = 128


def _pack_bf16_pairs(x):
    c = x.shape[1]
    bits = lax.bitcast_convert_type(x, jnp.uint32)
    rne = (bits + jnp.uint32(0x7FFF) + ((bits >> 16) & jnp.uint32(1))) >> 16
    return (rne[:, c // 2:] << 16) | rne[:, : c // 2]


def _unpack_bf16_pairs(w):
    lo = lax.bitcast_convert_type(w << 16, jnp.float32)
    hi = lax.bitcast_convert_type(w & jnp.uint32(0xFFFF0000), jnp.float32)
    return jnp.concatenate([lo, hi], axis=1)


def _dense_pre_body(ts_ref, len_ref, q_ref, m_ref, e_ref, temb_ref,
                    w_ref, bias_ref, a_ref, xt_ref, xs_ref,
                    out_ref, em1_ref, qt_ref, mt_ref):
    bidx = pl.program_id(0)
    H, C = xt_ref.shape[1], q_ref.shape[1]

    tq = q_ref[0].T
    tm = m_ref[0].T
    qt_ref[...] = _pack_bf16_pairs(tq)
    mt_ref[...] = _pack_bf16_pairs(tm)
    lv = len_ref[bidx]
    lf = lv.astype(jnp.float32)

    xt = xt_ref[bidx, :].reshape(H, 1)
    xs = xs_ref[bidx, :].reshape(H, 1)
    bf16 = jnp.bfloat16
    lane = lax.broadcasted_iota(jnp.int32, (H, C), 1)
    ohxt = (lane == xt).astype(bf16)
    ohxs = (lane == xs).astype(jnp.float32)

    emb = jnp.dot(ohxt, e_ref[...].astype(bf16),
                  preferred_element_type=jnp.float32)
    emb = emb + temb_ref[ts_ref[bidx], :][None, :]
    logits = (jnp.dot(emb.astype(bf16), w_ref[...].astype(bf16),
                      preferred_element_type=jnp.float32)
              + bias_ref[...][None, :])

    m = jnp.max(logits, axis=1, keepdims=True)
    ex = jnp.exp(logits - m)
    ssum = jnp.sum(ex, axis=1, keepdims=True)
    probs = ex / ssum

    lse = m + jnp.log(ssum)
    nll = -jnp.sum(ohxs * (logits - lse), axis=1, keepdims=True)
    pos = lax.broadcasted_iota(jnp.int32, (H, 1), 0)
    maskf = (pos < lv).astype(jnp.float32)
    ce_b = jnp.sum(nll * maskf) / lf

    em1 = lax.dot_general(probs.astype(bf16), tm.astype(bf16),
                          (((1,), (0,)), ((), ())),
                          preferred_element_type=jnp.float32)
    em1_ref[...] = _pack_bf16_pairs(em1)

    logp = jnp.log(probs + 1e-6)
    alogp = lax.dot_general(logp.astype(bf16), a_ref[...].astype(bf16),
                            (((1,), (1,)), ((), ())),
                            preferred_element_type=jnp.float32)
    pm = (pos[: H - 1] < lv - 1).astype(jnp.float32)
    q1 = jnp.sum(alogp[1:, :] * probs[: H - 1, :], axis=1, keepdims=True)
    q2 = jnp.sum(alogp[: H - 1, :] * probs[1:, :], axis=1, keepdims=True)
    t12 = (jnp.sum(q1 * pm) + jnp.sum(q2 * pm)) / ((lf - 1.0) * jnp.float32(C))

    lane_o = lax.broadcasted_iota(jnp.int32, (1, 1, 128), 2)
    out_ref[...] = jnp.where(
        lane_o == 1, ce_b, jnp.where(lane_o == 2, t12, 0.0))


_KL_SPLIT = 4


def _dense_kl_body(len_ref, etxt_ref, mtm1_ref, em1_ref, out_ref):
    gidx = pl.program_id(0)
    Hc = etxt_ref.shape[0]
    lv = len_ref[gidx // _KL_SPLIT]
    lf = lv.astype(jnp.float32)
    feps = jnp.float32(1.1920929e-07)

    etxt = _unpack_bf16_pairs(etxt_ref[...])
    pred_unorm = etxt * _unpack_bf16_pairs(em1_ref[...])
    s = jnp.maximum(jnp.sum(pred_unorm, axis=1, keepdims=True), 1e-8)
    pred_probs = jnp.clip(pred_unorm / s, feps, 1.0 - feps)

    true_unorm = etxt * _unpack_bf16_pairs(mtm1_ref[...])
    tp = true_unorm / jnp.sum(true_unorm, axis=1, keepdims=True)
    tp = jnp.clip(tp, feps, 1.0 - feps)
    kl_el = tp * (jnp.log(tp / pred_probs) - 1e-6)
    pos = (lax.broadcasted_iota(jnp.int32, (Hc, 1), 0)
           + (gidx % _KL_SPLIT) * Hc)
    maskf = (pos < lv).astype(jnp.float32)
    kl_b = jnp.sum(jnp.sum(kl_el, axis=1, keepdims=True) * maskf) / lf

    lane_o = lax.broadcasted_iota(jnp.int32, (1, 1, 128), 2)
    out_ref[...] = jnp.where(lane_o == 0, kl_b, 0.0)


@functools.lru_cache(maxsize=None)
def _make_sc_gather(B, H, C, rowspan):
    tok = B * H
    n_workers = _SC_CORES * _SC_SUBCORES
    per_w = tok // n_workers
    mesh = plsc.VectorSubcoreMesh(core_axis_name="c", subcore_axis_name="s")

    ch = _GATHER_CHUNK

    @functools.partial(
        pl.kernel,
        mesh=mesh,
        out_type=[jax.ShapeDtypeStruct((tok, C), jnp.uint32)] * 2,
        scratch_types=[
            pltpu.VMEM((per_w,), jnp.int32),
            pltpu.VMEM((per_w,), jnp.int32),
            pltpu.VMEM((ch, C), jnp.uint32),
            pltpu.VMEM((ch, C), jnp.uint32),
            pltpu.SemaphoreType.DMA,
            pltpu.SemaphoreType.DMA,
            pltpu.SemaphoreType.DMA,
            pltpu.SemaphoreType.DMA,
        ],
    )
    def gather_k(qt_hbm, mt_hbm, idxq_hbm, idxm_hbm, out_q, out_m,
                 idxq_v, idxm_v, rows0, rows1, g0, g1, w0, w1):
        wid = lax.axis_index("s") * _SC_CORES + lax.axis_index("c")
        base = wid * per_w
        bat = base // H
        col = base % H
        pltpu.sync_copy(idxq_hbm.at[bat, pl.ds(col, per_w)], idxq_v)
        pltpu.sync_copy(idxm_hbm.at[bat, pl.ds(col, per_w)], idxm_v)
        rowoff = bat * rowspan
        for j in range(0, per_w, 16):
            sl = pl.ds(j, 16)
            idxq_v[sl] = idxq_v[sl] + rowoff
            idxm_v[sl] = idxm_v[sl] + rowoff
        tasks = ([(qt_hbm, idxq_v, out_q, off)
                  for off in range(0, per_w, ch)]
                 + [(mt_hbm, idxm_v, out_m, off)
                    for off in range(0, per_w, ch)])
        bufs, gsems, wsems = (rows0, rows1), (g0, g1), (w0, w1)
        whandles = [None, None]
        for k, (tbl, idx_v, out_hbm, off) in enumerate(tasks):
            bi = k & 1
            if whandles[bi] is not None:
                whandles[bi].wait()
            gh = pltpu.async_copy(tbl.at[idx_v.at[pl.ds(off, ch)]],
                                  bufs[bi], gsems[bi])
            gh.wait()
            whandles[bi] = pltpu.async_copy(
                bufs[bi], out_hbm.at[pl.ds(base + off, ch)], wsems[bi])
        whandles[0].wait()
        whandles[1].wait()

    return gather_k


def kernel(matrices, Q, A, E, Temb, W, b, xs_padded, xt_padded, lengths, ts):
    B, H = xt_padded.shape
    C = Q.shape[1]
    D = E.shape[1]
    ts32 = ts.astype(jnp.int32)
    len32 = lengths.astype(jnp.int32)
    xt32 = xt_padded.astype(jnp.int32)
    xs32 = xs_padded.astype(jnp.int32)

    parts1, em1p, QT, MT = pl.pallas_call(
        _dense_pre_body,
        grid_spec=pltpu.PrefetchScalarGridSpec(
            num_scalar_prefetch=2,
            grid=(B,),
            in_specs=[
                pl.BlockSpec((1, C, C), lambda i, ts_r, ln: (ts_r[i], 0, 0)),
                pl.BlockSpec((1, C, C),
                             lambda i, ts_r, ln: (ts_r[i] - 1, 0, 0)),
                pl.BlockSpec((C, D), lambda i, ts_r, ln: (0, 0)),
                pl.BlockSpec(Temb.shape, lambda i, ts_r, ln: (0, 0)),
                pl.BlockSpec((D, C), lambda i, ts_r, ln: (0, 0)),
                pl.BlockSpec(b.shape, lambda i, ts_r, ln: (0,)),
                pl.BlockSpec((C, C), lambda i, ts_r, ln: (0, 0)),
                pl.BlockSpec((B, H), lambda i, ts_r, ln: (0, 0)),
                pl.BlockSpec((B, H), lambda i, ts_r, ln: (0, 0)),
            ],
            out_specs=[
                pl.BlockSpec((1, 1, 128), lambda i, ts_r, ln: (i, 0, 0)),
                pl.BlockSpec((H, C // 2), lambda i, ts_r, ln: (i, 0)),
                pl.BlockSpec((C, C // 2), lambda i, ts_r, ln: (i, 0)),
                pl.BlockSpec((C, C // 2), lambda i, ts_r, ln: (i, 0)),
            ],
        ),
        out_shape=[
            jax.ShapeDtypeStruct((B, 1, 128), jnp.float32),
            jax.ShapeDtypeStruct((B * H, C // 2), jnp.uint32),
            jax.ShapeDtypeStruct((B * C, C // 2), jnp.uint32),
            jax.ShapeDtypeStruct((B * C, C // 2), jnp.uint32),
        ],
    )(ts32, len32, Q, matrices, E, Temb, W, b, A, xt32, xs32)

    EtXt, Mtm1 = _make_sc_gather(B, H, C // 2, C)(QT, MT, xt32, xs32)

    ks = _KL_SPLIT
    parts2 = pl.pallas_call(
        _dense_kl_body,
        grid_spec=pltpu.PrefetchScalarGridSpec(
            num_scalar_prefetch=1,
            grid=(B * ks,),
            in_specs=[
                pl.BlockSpec((H // ks, C // 2), lambda i, ln: (i, 0)),
                pl.BlockSpec((H // ks, C // 2), lambda i, ln: (i, 0)),
                pl.BlockSpec((H // ks, C // 2), lambda i, ln: (i, 0)),
            ],
            out_specs=pl.BlockSpec((1, 1, 128), lambda i, ln: (i, 0, 0)),
        ),
        out_shape=jax.ShapeDtypeStruct((B * ks, 1, 128), jnp.float32),
    )(len32, EtXt, Mtm1, em1p)

    kl_loss = jnp.sum(parts2[:, 0, 0])
    ce_loss = jnp.sum(parts1[:, 0, 1])
    con_loss = -jnp.sum(parts1[:, 0, 2]) / jnp.float32(B)
    return (kl_loss, ce_loss, con_loss * 100.0)


# speedup vs baseline: 1.2645x; 1.2645x over previous
import functools

import jax
import jax.numpy as jnp
from jax import lax
from jax.experimental import pallas as pl
from jax.experimental.pallas import tpu as pltpu
from jax.experimental.pallas import tpu_sc as plsc

_SC_CORES = 2
_SC_SUBCORES = 16
_GATHER_CHUNK = 128


def _pack_bf16_pairs(x):
    c = x.shape[1]
    bits = lax.bitcast_convert_type(x, jnp.uint32)
    rne = (bits + jnp.uint32(0x7FFF) + ((bits >> 16) & jnp.uint32(1))) >> 16
    return (rne[:, c // 2:] << 16) | rne[:, : c // 2]


def _unpack_bf16_pairs(w):
    lo = lax.bitcast_convert_type(w << 16, jnp.float32)
    hi = lax.bitcast_convert_type(w & jnp.uint32(0xFFFF0000), jnp.float32)
    return jnp.concatenate([lo, hi], axis=1)


def _dense_pre_body(ts_ref, len_ref, q_ref, m_ref, e_ref, temb_ref,
                    w_ref, bias_ref, a_ref, xt_ref, xs_ref,
                    out_ref, em1_ref, qt_ref, mt_ref):
    bidx = pl.program_id(0)
    H, C = xt_ref.shape[1], q_ref.shape[1]

    tq = q_ref[0].T
    tm = m_ref[0].T
    qt_ref[...] = _pack_bf16_pairs(tq)
    mt_ref[...] = _pack_bf16_pairs(tm)
    lv = len_ref[bidx]
    lf = lv.astype(jnp.float32)

    xt = xt_ref[bidx, :].reshape(H, 1)
    xs = xs_ref[bidx, :].reshape(H, 1)
    bf16 = jnp.bfloat16
    lane = lax.broadcasted_iota(jnp.int32, (H, C), 1)
    ohxt = (lane == xt).astype(bf16)
    ohxs = (lane == xs).astype(jnp.float32)

    emb = jnp.dot(ohxt, e_ref[...].astype(bf16),
                  preferred_element_type=jnp.float32)
    emb = emb + temb_ref[ts_ref[bidx], :][None, :]
    logits = (jnp.dot(emb.astype(bf16), w_ref[...].astype(bf16),
                      preferred_element_type=jnp.float32)
              + bias_ref[...][None, :])

    m = jnp.max(logits, axis=1, keepdims=True)
    ex = jnp.exp(logits - m)
    ssum = jnp.sum(ex, axis=1, keepdims=True)
    probs = ex / ssum

    lse = m + jnp.log(ssum)
    nll = -jnp.sum(ohxs * (logits - lse), axis=1, keepdims=True)
    pos = lax.broadcasted_iota(jnp.int32, (H, 1), 0)
    maskf = (pos < lv).astype(jnp.float32)
    ce_b = jnp.sum(nll * maskf) / lf

    em1 = lax.dot_general(probs.astype(bf16), tm.astype(bf16),
                          (((1,), (0,)), ((), ())),
                          preferred_element_type=jnp.float32)
    em1_ref[...] = _pack_bf16_pairs(em1)

    logp = jnp.log(probs + 1e-6)
    alogp = lax.dot_general(logp.astype(bf16), a_ref[...].astype(bf16),
                            (((1,), (1,)), ((), ())),
                            preferred_element_type=jnp.float32)
    pm = (pos[: H - 1] < lv - 1).astype(jnp.float32)
    q1 = jnp.sum(alogp[1:, :] * probs[: H - 1, :], axis=1, keepdims=True)
    q2 = jnp.sum(alogp[: H - 1, :] * probs[1:, :], axis=1, keepdims=True)
    t12 = (jnp.sum(q1 * pm) + jnp.sum(q2 * pm)) / ((lf - 1.0) * jnp.float32(C))

    lane_o = lax.broadcasted_iota(jnp.int32, (1, 1, 128), 2)
    out_ref[...] = jnp.where(
        lane_o == 1, ce_b, jnp.where(lane_o == 2, t12, 0.0))


_KL_SPLIT = 1


def _dense_kl_body(len_ref, etxt_ref, mtm1_ref, em1_ref, out_ref):
    gidx = pl.program_id(0)
    Hc = etxt_ref.shape[0]
    lv = len_ref[gidx // _KL_SPLIT]
    lf = lv.astype(jnp.float32)
    feps = jnp.float32(1.1920929e-07)

    etxt = _unpack_bf16_pairs(etxt_ref[...])
    pred_unorm = etxt * _unpack_bf16_pairs(em1_ref[...])
    s = jnp.maximum(jnp.sum(pred_unorm, axis=1, keepdims=True), 1e-8)
    pred_probs = jnp.clip(pred_unorm / s, feps, 1.0 - feps)

    true_unorm = etxt * _unpack_bf16_pairs(mtm1_ref[...])
    tp = true_unorm / jnp.sum(true_unorm, axis=1, keepdims=True)
    tp = jnp.clip(tp, feps, 1.0 - feps)
    kl_el = tp * (jnp.log(tp / pred_probs) - 1e-6)
    pos = (lax.broadcasted_iota(jnp.int32, (Hc, 1), 0)
           + (gidx % _KL_SPLIT) * Hc)
    maskf = (pos < lv).astype(jnp.float32)
    kl_b = jnp.sum(jnp.sum(kl_el, axis=1, keepdims=True) * maskf) / lf

    lane_o = lax.broadcasted_iota(jnp.int32, (1, 1, 128), 2)
    out_ref[...] = jnp.where(lane_o == 0, kl_b, 0.0)


@functools.lru_cache(maxsize=None)
def _make_sc_gather(B, H, C, rowspan):
    tok = B * H
    n_workers = _SC_CORES * _SC_SUBCORES
    per_w = tok // n_workers
    mesh = plsc.VectorSubcoreMesh(core_axis_name="c", subcore_axis_name="s")

    ch = _GATHER_CHUNK

    @functools.partial(
        pl.kernel,
        mesh=mesh,
        out_type=[jax.ShapeDtypeStruct((tok, C), jnp.uint32)] * 2,
        scratch_types=[
            pltpu.VMEM((per_w,), jnp.int32),
            pltpu.VMEM((per_w,), jnp.int32),
            pltpu.VMEM((ch, C), jnp.uint32),
            pltpu.VMEM((ch, C), jnp.uint32),
            pltpu.SemaphoreType.DMA,
            pltpu.SemaphoreType.DMA,
            pltpu.SemaphoreType.DMA,
            pltpu.SemaphoreType.DMA,
        ],
    )
    def gather_k(qt_hbm, mt_hbm, idxq_hbm, idxm_hbm, out_q, out_m,
                 idxq_v, idxm_v, rows0, rows1, g0, g1, w0, w1):
        wid = lax.axis_index("s") * _SC_CORES + lax.axis_index("c")
        base = wid * per_w
        bat = base // H
        col = base % H
        pltpu.sync_copy(idxq_hbm.at[bat, pl.ds(col, per_w)], idxq_v)
        pltpu.sync_copy(idxm_hbm.at[bat, pl.ds(col, per_w)], idxm_v)
        rowoff = bat * rowspan
        for j in range(0, per_w, 16):
            sl = pl.ds(j, 16)
            idxq_v[sl] = idxq_v[sl] + rowoff
            idxm_v[sl] = idxm_v[sl] + rowoff
        tasks = ([(qt_hbm, idxq_v, out_q, off)
                  for off in range(0, per_w, ch)]
                 + [(mt_hbm, idxm_v, out_m, off)
                    for off in range(0, per_w, ch)])
        bufs, gsems, wsems = (rows0, rows1), (g0, g1), (w0, w1)
        whandles = [None, None]
        for k, (tbl, idx_v, out_hbm, off) in enumerate(tasks):
            bi = k & 1
            if whandles[bi] is not None:
                whandles[bi].wait()
            gh = pltpu.async_copy(tbl.at[idx_v.at[pl.ds(off, ch)]],
                                  bufs[bi], gsems[bi])
            gh.wait()
            whandles[bi] = pltpu.async_copy(
                bufs[bi], out_hbm.at[pl.ds(base + off, ch)], wsems[bi])
        whandles[0].wait()
        whandles[1].wait()

    return gather_k


def kernel(matrices, Q, A, E, Temb, W, b, xs_padded, xt_padded, lengths, ts):
    B, H = xt_padded.shape
    C = Q.shape[1]
    D = E.shape[1]
    ts32 = ts.astype(jnp.int32)
    len32 = lengths.astype(jnp.int32)
    xt32 = xt_padded.astype(jnp.int32)
    xs32 = xs_padded.astype(jnp.int32)

    parts1, em1p, QT, MT = pl.pallas_call(
        _dense_pre_body,
        grid_spec=pltpu.PrefetchScalarGridSpec(
            num_scalar_prefetch=2,
            grid=(B,),
            in_specs=[
                pl.BlockSpec((1, C, C), lambda i, ts_r, ln: (ts_r[i], 0, 0)),
                pl.BlockSpec((1, C, C),
                             lambda i, ts_r, ln: (ts_r[i] - 1, 0, 0)),
                pl.BlockSpec((C, D), lambda i, ts_r, ln: (0, 0)),
                pl.BlockSpec(Temb.shape, lambda i, ts_r, ln: (0, 0)),
                pl.BlockSpec((D, C), lambda i, ts_r, ln: (0, 0)),
                pl.BlockSpec(b.shape, lambda i, ts_r, ln: (0,)),
                pl.BlockSpec((C, C), lambda i, ts_r, ln: (0, 0)),
                pl.BlockSpec((B, H), lambda i, ts_r, ln: (0, 0)),
                pl.BlockSpec((B, H), lambda i, ts_r, ln: (0, 0)),
            ],
            out_specs=[
                pl.BlockSpec((1, 1, 128), lambda i, ts_r, ln: (i, 0, 0)),
                pl.BlockSpec((H, C // 2), lambda i, ts_r, ln: (i, 0)),
                pl.BlockSpec((C, C // 2), lambda i, ts_r, ln: (i, 0)),
                pl.BlockSpec((C, C // 2), lambda i, ts_r, ln: (i, 0)),
            ],
        ),
        out_shape=[
            jax.ShapeDtypeStruct((B, 1, 128), jnp.float32),
            jax.ShapeDtypeStruct((B * H, C // 2), jnp.uint32),
            jax.ShapeDtypeStruct((B * C, C // 2), jnp.uint32),
            jax.ShapeDtypeStruct((B * C, C // 2), jnp.uint32),
        ],
    )(ts32, len32, Q, matrices, E, Temb, W, b, A, xt32, xs32)

    EtXt, Mtm1 = _make_sc_gather(B, H, C // 2, C)(QT, MT, xt32, xs32)

    ks = _KL_SPLIT
    parts2 = pl.pallas_call(
        _dense_kl_body,
        grid_spec=pltpu.PrefetchScalarGridSpec(
            num_scalar_prefetch=1,
            grid=(B * ks,),
            in_specs=[
                pl.BlockSpec((H // ks, C // 2), lambda i, ln: (i, 0)),
                pl.BlockSpec((H // ks, C // 2), lambda i, ln: (i, 0)),
                pl.BlockSpec((H // ks, C // 2), lambda i, ln: (i, 0)),
            ],
            out_specs=pl.BlockSpec((1, 1, 128), lambda i, ln: (i, 0, 0)),
        ),
        out_shape=jax.ShapeDtypeStruct((B * ks, 1, 128), jnp.float32),
    )(len32, EtXt, Mtm1, em1p)

    kl_loss = jnp.sum(parts2[:, 0, 0])
    ce_loss = jnp.sum(parts1[:, 0, 1])
    con_loss = -jnp.sum(parts1[:, 0, 2]) / jnp.float32(B)
    return (kl_loss, ce_loss, con_loss * 100.0)
